# baseline (speedup 1.0000x reference)
import functools

import jax
import jax.numpy as jnp
from jax import lax
from jax.experimental import pallas as pl
from jax.experimental.pallas import tpu as pltpu
from jax.experimental.pallas import tpu_sc as plsc

N = 10000
D = 128
E = 320000

NC = 2
NS = 16
NW = NC * NS
CH = 64
NBUF = 4
G = 8
NBLK = 20
NCHUNK = G * NBLK
EPT = NCHUNK * CH
E_PAD = NW * EPT
NP = 10240
ROWS_PT = NP // NS
ZCH = 64
LANES = 16


def _sc_agg_body(h_hbm, src_hbm, dst_hbm, agg_hbm,
                 src_blk, dst_blk, r0, r1, r2, r3, agg_sh,
                 gsems, ssems, isem):
    rows = [r0, r1, r2, r3]
    c = lax.axis_index("c")
    s = lax.axis_index("s")
    w = c * NS + s

    pltpu.sync_copy(src_hbm.at[w, pl.ds(0, G)], src_blk.at[0])
    pltpu.sync_copy(dst_hbm.at[w, pl.ds(0, G)], dst_blk.at[0])

    def _zrow(r, carry):
        for cc in range(D // LANES):
            r0[r, pl.ds(cc * LANES, LANES)] = jnp.zeros((LANES,), jnp.float32)
        return carry
    lax.fori_loop(0, ZCH, _zrow, 0)
    for k in range(ROWS_PT // ZCH):
        pltpu.sync_copy(r0, agg_sh.at[pl.ds(s * ROWS_PT + k * ZCH, ZCH)])

    for b in range(NBUF - 1):
        pltpu.async_copy(h_hbm.at[src_blk.at[0, b]], rows[b], gsems.at[b])

    plsc.subcore_barrier()

    def _block(blk, carry):
        j0 = blk * G
        ph = blk % 2
        phn = 1 - ph
        for i in range(G):
            j = j0 + i
            b = i % NBUF
            pltpu.make_async_copy(h_hbm.at[src_blk.at[ph, i]],
                                  rows[b], gsems.at[b]).wait()
            pltpu.async_copy(rows[b], agg_sh.at[dst_blk.at[ph, i]],
                             ssems.at[b], add=True)
            bp = (b + NBUF - 1) % NBUF
            i_n = i + NBUF - 1

            @pl.when(j + NBUF - 1 < NCHUNK)
            def _refill():
                @pl.when(j >= 1)
                def _():
                    ip = i - 1
                    php, ipp = (ph, ip) if ip >= 0 else (phn, G - 1)
                    pltpu.make_async_copy(rows[bp],
                                          agg_sh.at[dst_blk.at[php, ipp]],
                                          ssems.at[bp]).wait()
                if i == 0:
                    @pl.when(blk + 1 < NBLK)
                    def _():
                        pltpu.async_copy(src_hbm.at[w, pl.ds((blk + 1) * G, G)],
                                         src_blk.at[phn], isem)
                        pltpu.async_copy(dst_hbm.at[w, pl.ds((blk + 1) * G, G)],
                                         dst_blk.at[phn], isem)
                if i_n == G:
                    pltpu.make_async_copy(src_hbm.at[w, pl.ds(0, G)],
                                          src_blk.at[phn], isem).wait()
                    pltpu.make_async_copy(dst_hbm.at[w, pl.ds(0, G)],
                                          dst_blk.at[phn], isem).wait()
                if i_n < G:
                    pltpu.async_copy(h_hbm.at[src_blk.at[ph, i_n]],
                                     rows[bp], gsems.at[bp])
                else:
                    pltpu.async_copy(h_hbm.at[src_blk.at[phn, i_n - G]],
                                     rows[bp], gsems.at[bp])
        return carry

    lax.fori_loop(0, NBLK, _block, 0)

    lph = (NBLK - 1) % 2
    for b in range(NBUF):
        i = G - NBUF + b
        pltpu.make_async_copy(rows[b], agg_sh.at[dst_blk.at[lph, i]],
                              ssems.at[b]).wait()

    plsc.subcore_barrier()
    for k in range(ROWS_PT // ZCH):
        base = s * ROWS_PT + k * ZCH
        pltpu.sync_copy(agg_sh.at[pl.ds(base, ZCH)],
                        agg_hbm.at[pl.ds(c * NP + base, ZCH)])


@functools.cache
def _sc_agg():
    return pl.kernel(
        _sc_agg_body,
        out_type=jax.ShapeDtypeStruct((NC * NP, D), jnp.float32),
        mesh=plsc.VectorSubcoreMesh(core_axis_name="c", subcore_axis_name="s",
                                    num_cores=NC, num_subcores=NS),
        scratch_types=(
            [pltpu.VMEM((2, G, CH), jnp.int32)] * 2 +
            [pltpu.VMEM((CH, D), jnp.float32)] * NBUF +
            [pltpu.VMEM_SHARED((NP, D), jnp.float32),
             pltpu.SemaphoreType.DMA((NBUF,)),
             pltpu.SemaphoreType.DMA((NBUF,)),
             pltpu.SemaphoreType.DMA]
        ),
    )


def _tc_layer_body(x_ref, agg_ref, w1_ref, b1_ref, al_ref, g_ref, be_ref,
                   w2_ref, b2_ref, o_ref):
    h = x_ref[...] + agg_ref[0, :N] + agg_ref[1, :N]
    h = jnp.dot(h, w1_ref[...], preferred_element_type=jnp.float32) + b1_ref[...]
    m = jnp.mean(h, axis=0, keepdims=True)
    o = h - al_ref[...] * m
    v = jnp.mean(o * o, axis=0, keepdims=True)
    h = g_ref[...] * o * lax.rsqrt(v + 1e-5) + be_ref[...]
    h = jnp.maximum(h, 0.0)
    h = jnp.dot(h, w2_ref[...], preferred_element_type=jnp.float32) + b2_ref[...]
    o_ref[...] = jnp.maximum(h, 0.0)


def _tc_layer(x, agg, w1, b1, al, g, be, w2, b2):
    return pl.pallas_call(
        _tc_layer_body,
        out_shape=jax.ShapeDtypeStruct((N, D), jnp.float32),
    )(x, agg.reshape(NC, NP, D), w1, b1.reshape(1, D), al.reshape(1, D),
      g.reshape(1, D), be.reshape(1, D), w2, b2.reshape(1, D))


def kernel(x, edge_index, W1_0, b1_0, alpha_0, gamma_0, beta_0, W2_0, b2_0,
           W1_1, b1_1, alpha_1, gamma_1, beta_1, W2_1, b2_1):
    pad = E_PAD - E
    src = jnp.concatenate([edge_index[0], jnp.zeros((pad,), jnp.int32)])
    dst = jnp.concatenate([edge_index[1], jnp.full((pad,), N, jnp.int32)])
    src = src.reshape(NW, NCHUNK, CH)
    dst = dst.reshape(NW, NCHUNK, CH)

    agg = _sc_agg()(x, src, dst)
    h = _tc_layer(x, agg, W1_0, b1_0, alpha_0, gamma_0, beta_0, W2_0, b2_0)
    agg = _sc_agg()(h, src, dst)
    h = _tc_layer(h, agg, W1_1, b1_1, alpha_1, gamma_1, beta_1, W2_1, b2_1)
    return h


# speedup vs baseline: 1.0950x; 1.0950x over previous
import functools

import jax
import jax.numpy as jnp
from jax import lax
from jax.experimental import pallas as pl
from jax.experimental.pallas import tpu as pltpu
from jax.experimental.pallas import tpu_sc as plsc

N = 10000
D = 128
E = 320000

NC = 2
NS = 16
NW = NC * NS
CH = 64
NBUF = 4
G = 8
NBLK = 20
NCHUNK = G * NBLK
EPT = NCHUNK * CH
E_PAD = NW * EPT
NP = 10240
ROWS_PT = NP // NS
ZCH = 64
LANES = 16


def _sc_agg_body(h_hbm, src_hbm, dst_hbm, agg_hbm,
                 src_blk, dst_blk, r0, r1, r2, r3, agg_sh,
                 gsems, ssems, isem):
    rows = [r0, r1, r2, r3]
    c = lax.axis_index("c")
    s = lax.axis_index("s")
    w = c * NS + s

    pltpu.sync_copy(src_hbm.at[w, pl.ds(0, G)], src_blk.at[0])
    pltpu.sync_copy(dst_hbm.at[w, pl.ds(0, G)], dst_blk.at[0])

    def _zrow(r, carry):
        for cc in range(D // LANES):
            r0[r, pl.ds(cc * LANES, LANES)] = jnp.zeros((LANES,), jnp.float32)
        return carry
    lax.fori_loop(0, ZCH, _zrow, 0)
    for k in range(ROWS_PT // ZCH):
        pltpu.sync_copy(r0, agg_sh.at[pl.ds(s * ROWS_PT + k * ZCH, ZCH)])

    for b in range(NBUF - 1):
        pltpu.async_copy(h_hbm.at[src_blk.at[0, b]], rows[b], gsems.at[b])

    plsc.subcore_barrier()

    def _block(blk, carry):
        j0 = blk * G
        ph = blk % 2
        phn = 1 - ph
        for i in range(G):
            j = j0 + i
            b = i % NBUF
            pltpu.make_async_copy(h_hbm.at[src_blk.at[ph, i]],
                                  rows[b], gsems.at[b]).wait()
            pltpu.async_copy(rows[b], agg_sh.at[dst_blk.at[ph, i]],
                             ssems.at[b], add=True)
            bp = (b + NBUF - 1) % NBUF
            i_n = i + NBUF - 1

            @pl.when(j + NBUF - 1 < NCHUNK)
            def _refill():
                @pl.when(j >= 1)
                def _():
                    ip = i - 1
                    php, ipp = (ph, ip) if ip >= 0 else (phn, G - 1)
                    pltpu.make_async_copy(rows[bp],
                                          agg_sh.at[dst_blk.at[php, ipp]],
                                          ssems.at[bp]).wait()
                if i == 0:
                    @pl.when(blk + 1 < NBLK)
                    def _():
                        pltpu.async_copy(src_hbm.at[w, pl.ds((blk + 1) * G, G)],
                                         src_blk.at[phn], isem)
                        pltpu.async_copy(dst_hbm.at[w, pl.ds((blk + 1) * G, G)],
                                         dst_blk.at[phn], isem)
                if i_n == G:
                    pltpu.make_async_copy(src_hbm.at[w, pl.ds(0, G)],
                                          src_blk.at[phn], isem).wait()
                    pltpu.make_async_copy(dst_hbm.at[w, pl.ds(0, G)],
                                          dst_blk.at[phn], isem).wait()
                if i_n < G:
                    pltpu.async_copy(h_hbm.at[src_blk.at[ph, i_n]],
                                     rows[bp], gsems.at[bp])
                else:
                    pltpu.async_copy(h_hbm.at[src_blk.at[phn, i_n - G]],
                                     rows[bp], gsems.at[bp])
        return carry

    lax.fori_loop(0, NBLK, _block, 0)

    lph = (NBLK - 1) % 2
    for b in range(NBUF):
        i = G - NBUF + b
        pltpu.make_async_copy(rows[b], agg_sh.at[dst_blk.at[lph, i]],
                              ssems.at[b]).wait()

    plsc.subcore_barrier()
    for k in range(ROWS_PT // ZCH):
        base = s * ROWS_PT + k * ZCH
        pltpu.sync_copy(agg_sh.at[pl.ds(base, ZCH)],
                        agg_hbm.at[pl.ds(c * NP + base, ZCH)])


@functools.cache
def _sc_agg():
    return pl.kernel(
        _sc_agg_body,
        out_type=jax.ShapeDtypeStruct((NC * NP, D), jnp.float32),
        mesh=plsc.VectorSubcoreMesh(core_axis_name="c", subcore_axis_name="s",
                                    num_cores=NC, num_subcores=NS),
        scratch_types=(
            [pltpu.VMEM((2, G, CH), jnp.int32)] * 2 +
            [pltpu.VMEM((CH, D), jnp.float32)] * NBUF +
            [pltpu.VMEM_SHARED((NP, D), jnp.float32),
             pltpu.SemaphoreType.DMA((NBUF,)),
             pltpu.SemaphoreType.DMA((NBUF,)),
             pltpu.SemaphoreType.DMA]
        ),
        compiler_params=pltpu.CompilerParams(use_tc_tiling_on_sc=False),
    )


def _tc_layer_body(x_ref, agg_ref, w1_ref, b1_ref, al_ref, g_ref, be_ref,
                   w2_ref, b2_ref, o_ref):
    h = x_ref[...] + agg_ref[0, :N] + agg_ref[1, :N]
    h = jnp.dot(h, w1_ref[...], preferred_element_type=jnp.float32) + b1_ref[...]
    m = jnp.mean(h, axis=0, keepdims=True)
    o = h - al_ref[...] * m
    v = jnp.mean(o * o, axis=0, keepdims=True)
    h = g_ref[...] * o * lax.rsqrt(v + 1e-5) + be_ref[...]
    h = jnp.maximum(h, 0.0)
    h = jnp.dot(h, w2_ref[...], preferred_element_type=jnp.float32) + b2_ref[...]
    o_ref[...] = jnp.maximum(h, 0.0)


def _tc_layer(x, agg, w1, b1, al, g, be, w2, b2):
    return pl.pallas_call(
        _tc_layer_body,
        out_shape=jax.ShapeDtypeStruct((N, D), jnp.float32),
    )(x, agg.reshape(NC, NP, D), w1, b1.reshape(1, D), al.reshape(1, D),
      g.reshape(1, D), be.reshape(1, D), w2, b2.reshape(1, D))


def kernel(x, edge_index, W1_0, b1_0, alpha_0, gamma_0, beta_0, W2_0, b2_0,
           W1_1, b1_1, alpha_1, gamma_1, beta_1, W2_1, b2_1):
    pad = E_PAD - E
    src = jnp.concatenate([edge_index[0], jnp.zeros((pad,), jnp.int32)])
    dst = jnp.concatenate([edge_index[1], jnp.full((pad,), N, jnp.int32)])
    src = src.reshape(NW, NCHUNK, CH)
    dst = dst.reshape(NW, NCHUNK, CH)

    agg = _sc_agg()(x, src, dst)
    h = _tc_layer(x, agg, W1_0, b1_0, alpha_0, gamma_0, beta_0, W2_0, b2_0)
    agg = _sc_agg()(h, src, dst)
    h = _tc_layer(h, agg, W1_1, b1_1, alpha_1, gamma_1, beta_1, W2_1, b2_1)
    return h


# speedup vs baseline: 3.9175x; 3.5776x over previous
import functools

import jax
import jax.numpy as jnp
from jax import lax
from jax.experimental import pallas as pl
from jax.experimental.pallas import tpu as pltpu
from jax.experimental.pallas import tpu_sc as plsc

N = 10000
D = 128
E = 320000
DH = D // 2

NC = 2
NS = 16
EPW = E // NS
CH = 125
NCHUNK = EPW // CH
NBUF = 5
NP = 10240
ROWS_PT = NP // NS
STAGE = 128
LANES = 16


def _sc_agg_body(h2_hbm, src_hbm, dst_hbm, agg_hbm,
                 src_v, dst_v, rows, stage_v, agg_sh, gsems, ssems):
    c = lax.axis_index("c")
    s = lax.axis_index("s")

    pltpu.sync_copy(src_hbm.at[s], src_v)
    pltpu.sync_copy(dst_hbm.at[s], dst_v)

    for b in range(NBUF - 1):
        pltpu.async_copy(h2_hbm.at[c].at[src_v.at[b]], rows.at[b], gsems.at[b])

    def _zrow(r, carry):
        for cc in range(DH // LANES):
            stage_v[r, pl.ds(cc * LANES, LANES)] = jnp.zeros((LANES,), jnp.float32)
        return carry
    lax.fori_loop(0, STAGE, _zrow, 0)
    for k in range(ROWS_PT // STAGE):
        pltpu.sync_copy(stage_v, agg_sh.at[pl.ds(s * ROWS_PT + k * STAGE, STAGE)])

    plsc.subcore_barrier()

    def _group(g, carry):
        j0 = g * NBUF
        for b in range(NBUF):
            j = j0 + b
            pltpu.make_async_copy(h2_hbm.at[c].at[src_v.at[j]],
                                  rows.at[b], gsems.at[b]).wait()
            pltpu.async_copy(rows.at[b], agg_sh.at[dst_v.at[j]], ssems.at[b],
                             add=True)
            bp = (b + NBUF - 1) % NBUF

            @pl.when(j + NBUF - 1 < NCHUNK)
            def _refill():
                @pl.when(j >= 1)
                def _():
                    pltpu.make_async_copy(rows.at[bp], agg_sh.at[dst_v.at[j - 1]],
                                          ssems.at[bp]).wait()
                pltpu.async_copy(h2_hbm.at[c].at[src_v.at[j + NBUF - 1]],
                                 rows.at[bp], gsems.at[bp])
        return carry

    lax.fori_loop(0, NCHUNK // NBUF, _group, 0)

    for b in range(NBUF):
        j = NCHUNK - NBUF + b
        pltpu.make_async_copy(rows.at[b], agg_sh.at[dst_v.at[j]],
                              ssems.at[b]).wait()

    plsc.subcore_barrier()
    for k in range(ROWS_PT // STAGE):
        base = s * ROWS_PT + k * STAGE
        pltpu.sync_copy(agg_sh.at[pl.ds(base, STAGE)],
                        agg_hbm.at[pl.ds(base, STAGE), pl.ds(c * DH, DH)])


@functools.cache
def _sc_agg():
    return pl.kernel(
        _sc_agg_body,
        out_type=jax.ShapeDtypeStruct((NP, D), jnp.float32),
        mesh=plsc.VectorSubcoreMesh(core_axis_name="c", subcore_axis_name="s",
                                    num_cores=NC, num_subcores=NS),
        scratch_types=[
            pltpu.VMEM((NCHUNK, CH), jnp.int32),
            pltpu.VMEM((NCHUNK, CH), jnp.int32),
            pltpu.VMEM((NBUF, CH, DH), jnp.float32),
            pltpu.VMEM((STAGE, DH), jnp.float32),
            pltpu.VMEM_SHARED((NP, DH), jnp.float32),
            pltpu.SemaphoreType.DMA((NBUF,)),
            pltpu.SemaphoreType.DMA((NBUF,)),
        ],
        compiler_params=pltpu.CompilerParams(use_tc_tiling_on_sc=False),
    )


def _tc_layer_body(x_ref, agg_ref, w1_ref, b1_ref, al_ref, g_ref, be_ref,
                   w2_ref, b2_ref, o_ref, o2_ref):
    h = x_ref[...] + agg_ref[:N]
    h = jnp.dot(h, w1_ref[...], preferred_element_type=jnp.float32) + b1_ref[...]
    m = jnp.mean(h, axis=0, keepdims=True)
    o = h - al_ref[...] * m
    v = jnp.mean(o * o, axis=0, keepdims=True)
    h = g_ref[...] * o * lax.rsqrt(v + 1e-5) + be_ref[...]
    h = jnp.maximum(h, 0.0)
    h = jnp.dot(h, w2_ref[...], preferred_element_type=jnp.float32) + b2_ref[...]
    h = jnp.maximum(h, 0.0)
    o_ref[...] = h
    o2_ref[0] = h[:, :DH]
    o2_ref[1] = h[:, DH:]


def _tc_layer(x, agg, w1, b1, al, g, be, w2, b2):
    return pl.pallas_call(
        _tc_layer_body,
        out_shape=(jax.ShapeDtypeStruct((N, D), jnp.float32),
                   jax.ShapeDtypeStruct((NC, N, DH), jnp.float32)),
    )(x, agg, w1, b1.reshape(1, D), al.reshape(1, D),
      g.reshape(1, D), be.reshape(1, D), w2, b2.reshape(1, D))


def kernel(x, edge_index, W1_0, b1_0, alpha_0, gamma_0, beta_0, W2_0, b2_0,
           W1_1, b1_1, alpha_1, gamma_1, beta_1, W2_1, b2_1):
    src = edge_index[0].reshape(NS, NCHUNK, CH)
    dst = edge_index[1].reshape(NS, NCHUNK, CH)
    x2 = jnp.stack([x[:, :DH], x[:, DH:]])

    agg = _sc_agg()(x2, src, dst)
    h, h2 = _tc_layer(x, agg, W1_0, b1_0, alpha_0, gamma_0, beta_0, W2_0, b2_0)
    agg = _sc_agg()(h2, src, dst)
    h, _ = _tc_layer(h, agg, W1_1, b1_1, alpha_1, gamma_1, beta_1, W2_1, b2_1)
    return h


# speedup vs baseline: 4.5356x; 1.1578x over previous
import functools

import jax
import jax.numpy as jnp
from jax import lax
from jax.experimental import pallas as pl
from jax.experimental.pallas import tpu as pltpu
from jax.experimental.pallas import tpu_sc as plsc

N = 10000
D = 128
E = 320000
DH = D // 2

NC = 2
NS = 16
EPW = E // NS
CH = 125
NCHUNK = EPW // CH
NBUF = 5
NP = 10240
ROWS_PT = NP // NS
STAGE = 128
LANES = 16


def _sc_agg_body(h2_hbm, src_hbm, dst_hbm, agg_hbm,
                 src_v, dst_v, rows, stage_v, agg_sh, gsems, ssems):
    c = lax.axis_index("c")
    s = lax.axis_index("s")

    pltpu.sync_copy(src_hbm.at[s], src_v)
    pltpu.sync_copy(dst_hbm.at[s], dst_v)

    h_half = h2_hbm.at[pl.ds(c, 2 * N - 1)]

    for b in range(NBUF - 1):
        pltpu.async_copy(h_half.at[src_v.at[b]], rows.at[b], gsems.at[b])

    def _zrow(r, carry):
        for cc in range(DH // LANES):
            stage_v[r, pl.ds(cc * LANES, LANES)] = jnp.zeros((LANES,), jnp.float32)
        return carry
    lax.fori_loop(0, STAGE, _zrow, 0)
    for k in range(ROWS_PT // STAGE):
        pltpu.sync_copy(stage_v, agg_sh.at[pl.ds(s * ROWS_PT + k * STAGE, STAGE)])

    plsc.subcore_barrier()

    def _group(g, carry):
        j0 = g * NBUF
        for b in range(NBUF):
            j = j0 + b
            pltpu.make_async_copy(h_half.at[src_v.at[j]],
                                  rows.at[b], gsems.at[b]).wait()
            pltpu.async_copy(rows.at[b], agg_sh.at[dst_v.at[j]], ssems.at[b],
                             add=True)
            bp = (b + NBUF - 1) % NBUF

            @pl.when(j + NBUF - 1 < NCHUNK)
            def _refill():
                @pl.when(j >= 1)
                def _():
                    pltpu.make_async_copy(rows.at[bp], agg_sh.at[dst_v.at[j - 1]],
                                          ssems.at[bp]).wait()
                pltpu.async_copy(h_half.at[src_v.at[j + NBUF - 1]],
                                 rows.at[bp], gsems.at[bp])
        return carry

    lax.fori_loop(0, NCHUNK // NBUF, _group, 0)

    for b in range(NBUF):
        j = NCHUNK - NBUF + b
        pltpu.make_async_copy(rows.at[b], agg_sh.at[dst_v.at[j]],
                              ssems.at[b]).wait()

    plsc.subcore_barrier()
    for k in range(ROWS_PT // STAGE):
        base = s * ROWS_PT + k * STAGE
        pltpu.sync_copy(agg_sh.at[pl.ds(base, STAGE)],
                        agg_hbm.at[pl.ds(base, STAGE), pl.ds(c * DH, DH)])


@functools.cache
def _sc_agg():
    return pl.kernel(
        _sc_agg_body,
        out_type=jax.ShapeDtypeStruct((NP, D), jnp.float32),
        mesh=plsc.VectorSubcoreMesh(core_axis_name="c", subcore_axis_name="s",
                                    num_cores=NC, num_subcores=NS),
        scratch_types=[
            pltpu.VMEM((NCHUNK, CH), jnp.int32),
            pltpu.VMEM((NCHUNK, CH), jnp.int32),
            pltpu.VMEM((NBUF, CH, DH), jnp.float32),
            pltpu.VMEM((STAGE, DH), jnp.float32),
            pltpu.VMEM_SHARED((NP, DH), jnp.float32),
            pltpu.SemaphoreType.DMA((NBUF,)),
            pltpu.SemaphoreType.DMA((NBUF,)),
        ],
        compiler_params=pltpu.CompilerParams(use_tc_tiling_on_sc=False),
    )


def _tc_layer_body(x_ref, agg_ref, w1_ref, b1_ref, al_ref, g_ref, be_ref,
                   w2_ref, b2_ref, o_ref):
    h = x_ref[...] + agg_ref[:N]
    h = jnp.dot(h, w1_ref[...], preferred_element_type=jnp.float32) + b1_ref[...]
    m = jnp.mean(h, axis=0, keepdims=True)
    o = h - al_ref[...] * m
    v = jnp.mean(o * o, axis=0, keepdims=True)
    h = g_ref[...] * o * lax.rsqrt(v + 1e-5) + be_ref[...]
    h = jnp.maximum(h, 0.0)
    h = jnp.dot(h, w2_ref[...], preferred_element_type=jnp.float32) + b2_ref[...]
    o_ref[...] = jnp.maximum(h, 0.0)


def _tc_layer(x, agg, w1, b1, al, g, be, w2, b2):
    return pl.pallas_call(
        _tc_layer_body,
        out_shape=jax.ShapeDtypeStruct((N, D), jnp.float32),
    )(x, agg, w1, b1.reshape(1, D), al.reshape(1, D),
      g.reshape(1, D), be.reshape(1, D), w2, b2.reshape(1, D))


def kernel(x, edge_index, W1_0, b1_0, alpha_0, gamma_0, beta_0, W2_0, b2_0,
           W1_1, b1_1, alpha_1, gamma_1, beta_1, W2_1, b2_1):
    src = (edge_index[0] * 2).reshape(NS, NCHUNK, CH)
    dst = edge_index[1].reshape(NS, NCHUNK, CH)

    agg = _sc_agg()(x.reshape(2 * N, DH), src, dst)
    h = _tc_layer(x, agg, W1_0, b1_0, alpha_0, gamma_0, beta_0, W2_0, b2_0)
    agg = _sc_agg()(h.reshape(2 * N, DH), src, dst)
    h = _tc_layer(h, agg, W1_1, b1_1, alpha_1, gamma_1, beta_1, W2_1, b2_1)
    return h
